# baseline (speedup 1.0000x reference)
import functools

import jax
import jax.numpy as jnp
from jax import lax
from jax.experimental import pallas as pl
from jax.experimental.pallas import tpu as pltpu
from jax.experimental.pallas import tpu_sc as plsc

_BITS = 48
_C = 100
_N = 50000
_BATCH = 128
_LAM = 1.0
_ETA_MU = 0.1
_T = 6400
_NT = -(-_N // _T)
_HI = lax.Precision.HIGHEST
_F32 = jnp.float32
_BF16 = jnp.bfloat16

_INTERPRET = False


def _eye(n):
    ii = lax.broadcasted_iota(jnp.int32, (n, n), 0)
    jj = lax.broadcasted_iota(jnp.int32, (n, n), 1)
    return (ii == jj).astype(_F32)


def _row_set(M, i, row):
    ii = lax.broadcasted_iota(jnp.int32, (M.shape[0], 1), 0)
    return jnp.where(ii == i, row, M)


def _gj_solve(A, R):
    M = jnp.concatenate([A, R], axis=1)
    for i in range(_BITS):
        piv = M[i:i + 1, i:i + 1]
        row = M[i:i + 1, :] / piv
        col = M[:, i:i + 1]
        M = M - col * row
        M = _row_set(M, i, row)
    return M[:, _BITS:]


def _zero_diag(G):
    return G * (1.0 - _eye(_BITS))


def _bit_loop(B_scr, Q_scr, P, G0):
    Q_scr[...] = lax.dot_general(G0, B_scr[...], (((1,), (0,)), ((), ())),
                                 precision=_HI)
    for i in range(_BITS):
        newb = jnp.sign(P[i:i + 1, :] - Q_scr[i:i + 1, :])
        delta = newb - B_scr[i:i + 1, :]
        B_scr[i:i + 1, :] = newb
        Q_scr[...] = Q_scr[...] + G0[:, i:i + 1] * delta
    return B_scr[...]


def _onehot_from_labels(lab):
    cc = lax.broadcasted_iota(jnp.int32, (_C, 1), 0).astype(_F32)
    return (cc == lab).astype(_F32)


def _colmask(step, t):
    col = lax.broadcasted_iota(jnp.int32, (1, t), 1) + step * t
    return col < _N


def _k1_body(B_ref, Y_ref, S0_ref, R0_ref, lab_ref):
    @pl.when(pl.program_id(0) == 0)
    def _():
        S0_ref[...] = jnp.zeros_like(S0_ref)
        R0_ref[...] = jnp.zeros_like(R0_ref)

    mask = _colmask(pl.program_id(0), _T)
    bb = jnp.where(mask, B_ref[...], 0.0).astype(_BF16)
    yy = jnp.where(mask, Y_ref[...], 0.0).astype(_BF16)
    S0_ref[...] += lax.dot_general(bb, bb, (((1,), (1,)), ((), ())),
                                   preferred_element_type=_F32)
    R0_ref[...] += lax.dot_general(bb, yy, (((1,), (1,)), ((), ())),
                                   preferred_element_type=_F32)
    cvec = lax.broadcasted_iota(jnp.int32, (1, _C), 1).astype(_BF16)
    lab_ref[...] = lax.dot_general(cvec, yy, (((1,), (0,)), ((), ())),
                                   preferred_element_type=_F32)


def _k1_call(B, Y):
    return pl.pallas_call(
        _k1_body,
        grid=(_NT,),
        in_specs=[
            pl.BlockSpec((_BITS, _T), lambda s: (0, s)),
            pl.BlockSpec((_C, _T), lambda s: (0, s)),
        ],
        out_specs=[
            pl.BlockSpec((_BITS, _BITS), lambda s: (0, 0)),
            pl.BlockSpec((_BITS, _C), lambda s: (0, 0)),
            pl.BlockSpec((1, _T), lambda s: (0, s)),
        ],
        out_shape=[
            jax.ShapeDtypeStruct((_BITS, _BITS), _F32),
            jax.ShapeDtypeStruct((_BITS, _C), _F32),
            jax.ShapeDtypeStruct((1, _N), _F32),
        ],
        interpret=_INTERPRET,
    )(B, Y)


def _k2_body(S0_ref, R0_ref, B_ref, U_ref, lab_ref,
             S1_ref, R1_ref, W1_ref, W_scr, G0_scr, B_scr, Q_scr):
    @pl.when(pl.program_id(0) == 0)
    def _():
        A = S0_ref[...] + _LAM * _eye(_BITS)
        W = _gj_solve(A, R0_ref[...])
        G = lax.dot_general(W, W, (((1,), (1,)), ((), ())), precision=_HI)
        G0_scr[...] = _zero_diag(G)
        W_scr[...] = W
        W1_ref[...] = W
        S1_ref[...] = jnp.zeros_like(S1_ref)
        R1_ref[...] = jnp.zeros_like(R1_ref)

    W = W_scr[...]
    G0 = G0_scr[...]
    mask = _colmask(pl.program_id(0), _T)
    oneh = jnp.where(mask, _onehot_from_labels(lab_ref[...]), 0.0)
    P = lax.dot_general(W, oneh, (((1,), (0,)), ((), ())),
                        precision=_HI) + _ETA_MU * U_ref[...]
    B_scr[...] = B_ref[...]
    Bn = _bit_loop(B_scr, Q_scr, P, G0)
    bb = jnp.where(mask, Bn, 0.0).astype(_BF16)
    ob = oneh.astype(_BF16)
    S1_ref[...] += lax.dot_general(bb, bb, (((1,), (1,)), ((), ())),
                                   preferred_element_type=_F32)
    R1_ref[...] += lax.dot_general(bb, ob, (((1,), (1,)), ((), ())),
                                   preferred_element_type=_F32)


def _k2_call(S0, R0, B, U, labels):
    return pl.pallas_call(
        _k2_body,
        grid=(_NT,),
        in_specs=[
            pl.BlockSpec((_BITS, _BITS), lambda s: (0, 0)),
            pl.BlockSpec((_BITS, _C), lambda s: (0, 0)),
            pl.BlockSpec((_BITS, _T), lambda s: (0, s)),
            pl.BlockSpec((_BITS, _T), lambda s: (0, s)),
            pl.BlockSpec((1, _T), lambda s: (0, s)),
        ],
        out_specs=[
            pl.BlockSpec((_BITS, _BITS), lambda s: (0, 0)),
            pl.BlockSpec((_BITS, _C), lambda s: (0, 0)),
            pl.BlockSpec((_BITS, _C), lambda s: (0, 0)),
        ],
        out_shape=[
            jax.ShapeDtypeStruct((_BITS, _BITS), _F32),
            jax.ShapeDtypeStruct((_BITS, _C), _F32),
            jax.ShapeDtypeStruct((_BITS, _C), _F32),
        ],
        scratch_shapes=[
            pltpu.VMEM((_BITS, _C), _F32),
            pltpu.VMEM((_BITS, _BITS), _F32),
            pltpu.VMEM((_BITS, _T), _F32),
            pltpu.VMEM((_BITS, _T), _F32),
        ],
        interpret=_INTERPRET,
    )(S0, R0, B, U, labels)


def _k3_body(Ub_ref, Yb_ref, S1_ref, R1_ref, W1_ref, Bi_ref, Ui_ref, li_ref,
             l_ref, sl_ref, cl_ref, ql_ref, B_scr, Q_scr):
    W1 = W1_ref[...]
    A = S1_ref[...] + _LAM * _eye(_BITS)
    W2 = _gj_solve(A, R1_ref[...])
    G10 = _zero_diag(lax.dot_general(W1, W1, (((1,), (1,)), ((), ())),
                                     precision=_HI))
    G20 = _zero_diag(lax.dot_general(W2, W2, (((1,), (1,)), ((), ())),
                                     precision=_HI))
    oneh = _onehot_from_labels(li_ref[...])
    Ui = Ui_ref[...]
    P1 = lax.dot_general(W1, oneh, (((1,), (0,)), ((), ())),
                         precision=_HI) + _ETA_MU * Ui
    B_scr[...] = Bi_ref[...]
    _bit_loop(B_scr, Q_scr, P1, G10)
    P2 = lax.dot_general(W2, oneh, (((1,), (0,)), ((), ())),
                         precision=_HI) + _ETA_MU * Ui
    B2 = _bit_loop(B_scr, Q_scr, P2, G20)

    Ub = Ub_ref[...]
    Yb = Yb_ref[...]
    theta = 0.5 * lax.dot_general(Ub, Ub, (((1,), (1,)), ((), ())),
                                  precision=_HI)
    yb16 = Yb.astype(_BF16)
    Sm = (lax.dot_general(yb16, yb16, (((1,), (1,)), ((), ())),
                          preferred_element_type=_F32) > 0).astype(_F32)
    sp = jnp.maximum(theta, 0.0) + jnp.log(1.0 + jnp.exp(-jnp.abs(theta)))
    sim = jnp.mean(sp - Sm * theta)

    WB_T = lax.dot_general(B2, W2, (((0,), (0,)), ((), ())),
                           precision=_HI)
    cls = jnp.mean((Yb - WB_T) ** 2)
    qua = jnp.mean((Ub - jnp.transpose(B2)) ** 2)

    l_ref[...] = jnp.reshape(sim + 1.0 * cls + _ETA_MU * qua, (1, 1))
    sl_ref[...] = jnp.reshape(sim, (1, 1))
    cl_ref[...] = jnp.reshape(cls, (1, 1))
    ql_ref[...] = jnp.reshape(qua, (1, 1))


def _k3_call(Ub, Yb, S1, R1, W1, Bi, Ui, li):
    return pl.pallas_call(
        _k3_body,
        out_shape=[jax.ShapeDtypeStruct((1, 1), _F32)] * 4,
        scratch_shapes=[
            pltpu.VMEM((_BITS, _BATCH), _F32),
            pltpu.VMEM((_BITS, _BATCH), _F32),
        ],
        interpret=_INTERPRET,
    )(Ub, Yb, S1, R1, W1, Bi, Ui, li)


_NW = 32
_NCB = _BATCH // 16
_RPG = _BITS // (_NW // _NCB)


def _sc_gather(b_flat, u_flat, lab_flat, indices):
    mesh = plsc.VectorSubcoreMesh(core_axis_name="c", subcore_axis_name="s")

    @functools.partial(
        pl.kernel,
        mesh=mesh,
        out_type=(
            jax.ShapeDtypeStruct((_BITS * _BATCH,), _F32),
            jax.ShapeDtypeStruct((_BITS * _BATCH,), _F32),
            jax.ShapeDtypeStruct((_BATCH,), _F32),
        ),
        scratch_types=[
            pltpu.VMEM((16,), jnp.int32),
            pltpu.VMEM((_RPG, 16), _F32),
            pltpu.VMEM((16,), _F32),
            pltpu.SemaphoreType.DMA,
        ],
    )
    def k(b_hbm, u_hbm, lab_hbm, idx_hbm, bi_hbm, ui_hbm, li_hbm,
          idx_v, rbuf, lbuf, sem):
        wid = lax.axis_index("s") * 2 + lax.axis_index("c")
        cb = wid % _NCB
        rg = wid // _NCB
        r0 = rg * _RPG
        pltpu.sync_copy(idx_hbm.at[pl.ds(cb * 16, 16)], idx_v)
        ivec = idx_v[...]
        for src, dst in ((b_hbm, bi_hbm), (u_hbm, ui_hbm)):
            handles = []
            for r in range(_RPG):
                flat = ivec + (r0 + r) * _N
                handles.append(pltpu.async_copy(src.at[flat], rbuf.at[r], sem))
            for h in handles:
                h.wait()
            for r in range(_RPG):
                pltpu.sync_copy(
                    rbuf.at[r],
                    dst.at[pl.ds((r0 + r) * _BATCH + cb * 16, 16)])

        @pl.when(rg == 0)
        def _():
            pltpu.async_copy(lab_hbm.at[ivec], lbuf, sem).wait()
            pltpu.sync_copy(lbuf, li_hbm.at[pl.ds(cb * 16, 16)])

    return k(b_flat, u_flat, lab_flat, indices)


def kernel(image_hash_features, onehot_labels, indices, B, U, Y):
    S0, R0, labels = _k1_call(B, Y)
    Bi, Ui, lif = _sc_gather(B.reshape(-1), U.reshape(-1),
                             labels.reshape(-1), indices)
    S1, R1, W1 = _k2_call(S0, R0, B, U, labels)
    l, sl, cl, ql = _k3_call(image_hash_features, onehot_labels,
                             S1, R1, W1, Bi.reshape(_BITS, _BATCH),
                             Ui.reshape(_BITS, _BATCH),
                             lif.reshape(1, _BATCH))
    return (l[0, 0], sl[0, 0], cl[0, 0], ql[0, 0])


# speedup vs baseline: 1.3631x; 1.3631x over previous
import functools

import jax
import jax.numpy as jnp
from jax import lax
from jax.experimental import pallas as pl
from jax.experimental.pallas import tpu as pltpu
from jax.experimental.pallas import tpu_sc as plsc

_BITS = 48
_C = 100
_N = 50000
_BATCH = 128
_LAM = 1.0
_ETA_MU = 0.1
_T = 6400
_NT = -(-_N // _T)
_HI = lax.Precision.HIGHEST
_F32 = jnp.float32
_BF16 = jnp.bfloat16

_INTERPRET = False


def _eye(n):
    ii = lax.broadcasted_iota(jnp.int32, (n, n), 0)
    jj = lax.broadcasted_iota(jnp.int32, (n, n), 1)
    return (ii == jj).astype(_F32)


def _row_set(M, i, row):
    ii = lax.broadcasted_iota(jnp.int32, (M.shape[0], 1), 0)
    return jnp.where(ii == i, row, M)


def _gj_solve(A, R):
    M = jnp.concatenate([A, R], axis=1)
    for i in range(_BITS):
        piv = M[i:i + 1, i:i + 1]
        row = M[i:i + 1, :] / piv
        col = M[:, i:i + 1]
        M = M - col * row
        M = _row_set(M, i, row)
    return M[:, _BITS:]


def _zero_diag(G):
    return G * (1.0 - _eye(_BITS))


_BS = 8


def _bit_loop(B_scr, P, G0):
    G16 = G0.astype(_BF16)
    for b0 in range(0, _BITS, _BS):
        Qb = lax.dot_general(G16[b0:b0 + _BS, :], B_scr[...],
                             (((1,), (0,)), ((), ())),
                             preferred_element_type=_F32)
        deltas = []
        for li in range(_BS):
            i = b0 + li
            x = P[i:i + 1, :] - Qb[li:li + 1, :]
            for j in range(li):
                x = x - G0[b0 + j:b0 + j + 1, i:i + 1] * deltas[j]
            old = B_scr[i:i + 1, :].astype(_F32)
            newb = jnp.where(x > 0.0, 1.0, -1.0)
            deltas.append(newb - old)
            B_scr[i:i + 1, :] = newb.astype(_BF16)
    return B_scr[...]


def _onehot_from_labels(lab):
    cc = lax.broadcasted_iota(jnp.int32, (_C, 1), 0).astype(_F32)
    return (cc == lab).astype(_F32)


def _colmask(step, t):
    col = lax.broadcasted_iota(jnp.int32, (1, t), 1) + step * t
    return col < _N


def _k1_body(B_ref, Y_ref, S0_ref, R0_ref, lab_ref):
    @pl.when(pl.program_id(0) == 0)
    def _():
        S0_ref[...] = jnp.zeros_like(S0_ref)
        R0_ref[...] = jnp.zeros_like(R0_ref)

    mask = _colmask(pl.program_id(0), _T)
    bb = jnp.where(mask, B_ref[...], 0.0).astype(_BF16)
    yy = jnp.where(mask, Y_ref[...], 0.0).astype(_BF16)
    S0_ref[...] += lax.dot_general(bb, bb, (((1,), (1,)), ((), ())),
                                   preferred_element_type=_F32)
    R0_ref[...] += lax.dot_general(bb, yy, (((1,), (1,)), ((), ())),
                                   preferred_element_type=_F32)
    cvec = lax.broadcasted_iota(jnp.int32, (1, _C), 1).astype(_BF16)
    lab_ref[...] = lax.dot_general(cvec, yy, (((1,), (0,)), ((), ())),
                                   preferred_element_type=_F32)


def _k1_call(B, Y):
    return pl.pallas_call(
        _k1_body,
        grid=(_NT,),
        in_specs=[
            pl.BlockSpec((_BITS, _T), lambda s: (0, s)),
            pl.BlockSpec((_C, _T), lambda s: (0, s)),
        ],
        out_specs=[
            pl.BlockSpec((_BITS, _BITS), lambda s: (0, 0)),
            pl.BlockSpec((_BITS, _C), lambda s: (0, 0)),
            pl.BlockSpec((1, _T), lambda s: (0, s)),
        ],
        out_shape=[
            jax.ShapeDtypeStruct((_BITS, _BITS), _F32),
            jax.ShapeDtypeStruct((_BITS, _C), _F32),
            jax.ShapeDtypeStruct((1, _N), _F32),
        ],
        interpret=_INTERPRET,
    )(B, Y)


def _k2_body(S0_ref, R0_ref, B_ref, U_ref, lab_ref,
             S1_ref, R1_ref, W1_ref, W_scr, G0_scr, B_scr):
    @pl.when(pl.program_id(0) == 0)
    def _():
        A = S0_ref[...] + _LAM * _eye(_BITS)
        W = _gj_solve(A, R0_ref[...])
        G = lax.dot_general(W, W, (((1,), (1,)), ((), ())), precision=_HI)
        G0_scr[...] = _zero_diag(G)
        W_scr[...] = W
        W1_ref[...] = W
        S1_ref[...] = jnp.zeros_like(S1_ref)
        R1_ref[...] = jnp.zeros_like(R1_ref)

    W = W_scr[...]
    G0 = G0_scr[...]
    mask = _colmask(pl.program_id(0), _T)
    oneh = jnp.where(mask, _onehot_from_labels(lab_ref[...]), 0.0)
    ob = oneh.astype(_BF16)
    P = lax.dot_general(W.astype(_BF16), ob, (((1,), (0,)), ((), ())),
                        preferred_element_type=_F32) + _ETA_MU * U_ref[...]
    B_scr[...] = B_ref[...].astype(_BF16)
    Bn = _bit_loop(B_scr, P, G0)
    bb = jnp.where(mask, Bn, jnp.bfloat16(0))
    S1_ref[...] += lax.dot_general(bb, bb, (((1,), (1,)), ((), ())),
                                   preferred_element_type=_F32)
    R1_ref[...] += lax.dot_general(bb, ob, (((1,), (1,)), ((), ())),
                                   preferred_element_type=_F32)


def _k2_call(S0, R0, B, U, labels):
    return pl.pallas_call(
        _k2_body,
        grid=(_NT,),
        in_specs=[
            pl.BlockSpec((_BITS, _BITS), lambda s: (0, 0)),
            pl.BlockSpec((_BITS, _C), lambda s: (0, 0)),
            pl.BlockSpec((_BITS, _T), lambda s: (0, s)),
            pl.BlockSpec((_BITS, _T), lambda s: (0, s)),
            pl.BlockSpec((1, _T), lambda s: (0, s)),
        ],
        out_specs=[
            pl.BlockSpec((_BITS, _BITS), lambda s: (0, 0)),
            pl.BlockSpec((_BITS, _C), lambda s: (0, 0)),
            pl.BlockSpec((_BITS, _C), lambda s: (0, 0)),
        ],
        out_shape=[
            jax.ShapeDtypeStruct((_BITS, _BITS), _F32),
            jax.ShapeDtypeStruct((_BITS, _C), _F32),
            jax.ShapeDtypeStruct((_BITS, _C), _F32),
        ],
        scratch_shapes=[
            pltpu.VMEM((_BITS, _C), _F32),
            pltpu.VMEM((_BITS, _BITS), _F32),
            pltpu.VMEM((_BITS, _T), _BF16),
        ],
        interpret=_INTERPRET,
    )(S0, R0, B, U, labels)


def _k3_body(Ub_ref, Yb_ref, S1_ref, R1_ref, W1_ref, Bi_ref, Ui_ref, li_ref,
             l_ref, sl_ref, cl_ref, ql_ref, B_scr):
    W1 = W1_ref[...]
    A = S1_ref[...] + _LAM * _eye(_BITS)
    W2 = _gj_solve(A, R1_ref[...])
    G10 = _zero_diag(lax.dot_general(W1, W1, (((1,), (1,)), ((), ())),
                                     precision=_HI))
    G20 = _zero_diag(lax.dot_general(W2, W2, (((1,), (1,)), ((), ())),
                                     precision=_HI))
    oneh = _onehot_from_labels(li_ref[...])
    Ui = Ui_ref[...]
    P1 = lax.dot_general(W1, oneh, (((1,), (0,)), ((), ())),
                         precision=_HI) + _ETA_MU * Ui
    B_scr[...] = Bi_ref[...].astype(_BF16)
    _bit_loop(B_scr, P1, G10)
    P2 = lax.dot_general(W2, oneh, (((1,), (0,)), ((), ())),
                         precision=_HI) + _ETA_MU * Ui
    B2 = _bit_loop(B_scr, P2, G20).astype(_F32)

    Ub = Ub_ref[...]
    Yb = Yb_ref[...]
    theta = 0.5 * lax.dot_general(Ub, Ub, (((1,), (1,)), ((), ())),
                                  precision=_HI)
    yb16 = Yb.astype(_BF16)
    Sm = (lax.dot_general(yb16, yb16, (((1,), (1,)), ((), ())),
                          preferred_element_type=_F32) > 0).astype(_F32)
    sp = jnp.maximum(theta, 0.0) + jnp.log(1.0 + jnp.exp(-jnp.abs(theta)))
    sim = jnp.mean(sp - Sm * theta)

    WB_T = lax.dot_general(B2, W2, (((0,), (0,)), ((), ())),
                           precision=_HI)
    cls = jnp.mean((Yb - WB_T) ** 2)
    qua = jnp.mean((Ub - jnp.transpose(B2)) ** 2)

    l_ref[...] = jnp.reshape(sim + 1.0 * cls + _ETA_MU * qua, (1, 1))
    sl_ref[...] = jnp.reshape(sim, (1, 1))
    cl_ref[...] = jnp.reshape(cls, (1, 1))
    ql_ref[...] = jnp.reshape(qua, (1, 1))


def _k3_call(Ub, Yb, S1, R1, W1, Bi, Ui, li):
    return pl.pallas_call(
        _k3_body,
        out_shape=[jax.ShapeDtypeStruct((1, 1), _F32)] * 4,
        scratch_shapes=[
            pltpu.VMEM((_BITS, _BATCH), _BF16),
        ],
        interpret=_INTERPRET,
    )(Ub, Yb, S1, R1, W1, Bi, Ui, li)


_NW = 32
_NCB = _BATCH // 16
_RPG = _BITS // (_NW // _NCB)


def _sc_gather(b_flat, u_flat, lab_flat, indices):
    mesh = plsc.VectorSubcoreMesh(core_axis_name="c", subcore_axis_name="s")

    @functools.partial(
        pl.kernel,
        mesh=mesh,
        out_type=(
            jax.ShapeDtypeStruct((_BITS * _BATCH,), _F32),
            jax.ShapeDtypeStruct((_BITS * _BATCH,), _F32),
            jax.ShapeDtypeStruct((_BATCH,), _F32),
        ),
        scratch_types=[
            pltpu.VMEM((16,), jnp.int32),
            pltpu.VMEM((_RPG, 16), _F32),
            pltpu.VMEM((16,), _F32),
            pltpu.SemaphoreType.DMA,
        ],
    )
    def k(b_hbm, u_hbm, lab_hbm, idx_hbm, bi_hbm, ui_hbm, li_hbm,
          idx_v, rbuf, lbuf, sem):
        wid = lax.axis_index("s") * 2 + lax.axis_index("c")
        cb = wid % _NCB
        rg = wid // _NCB
        r0 = rg * _RPG
        pltpu.sync_copy(idx_hbm.at[pl.ds(cb * 16, 16)], idx_v)
        ivec = idx_v[...]
        for src, dst in ((b_hbm, bi_hbm), (u_hbm, ui_hbm)):
            handles = []
            for r in range(_RPG):
                flat = ivec + (r0 + r) * _N
                handles.append(pltpu.async_copy(src.at[flat], rbuf.at[r], sem))
            for h in handles:
                h.wait()
            for r in range(_RPG):
                pltpu.sync_copy(
                    rbuf.at[r],
                    dst.at[pl.ds((r0 + r) * _BATCH + cb * 16, 16)])

        @pl.when(rg == 0)
        def _():
            pltpu.async_copy(lab_hbm.at[ivec], lbuf, sem).wait()
            pltpu.sync_copy(lbuf, li_hbm.at[pl.ds(cb * 16, 16)])

    return k(b_flat, u_flat, lab_flat, indices)


def kernel(image_hash_features, onehot_labels, indices, B, U, Y):
    S0, R0, labels = _k1_call(B, Y)
    Bi, Ui, lif = _sc_gather(B.reshape(-1), U.reshape(-1),
                             labels.reshape(-1), indices)
    S1, R1, W1 = _k2_call(S0, R0, B, U, labels)
    l, sl, cl, ql = _k3_call(image_hash_features, onehot_labels,
                             S1, R1, W1, Bi.reshape(_BITS, _BATCH),
                             Ui.reshape(_BITS, _BATCH),
                             lif.reshape(1, _BATCH))
    return (l[0, 0], sl[0, 0], cl[0, 0], ql[0, 0])
